# baseline (speedup 1.0000x reference)
import functools
import jax
import jax.numpy as jnp
from jax import lax
from jax.experimental import pallas as pl
from jax.experimental.pallas import tpu as pltpu
from jax.experimental.pallas import tpu_sc as plsc

_C = 128
_S = 8192
_RT = 6144
_NW = 32
_RPW = (_S - _RT) // _NW


def _xchg_axis0(A, m, k, ig, ig0):
    n0 = A.shape[0]
    up = jnp.concatenate([A[m:], A[:m]], axis=0)
    dn = jnp.concatenate([A[n0 - m:], A[:n0 - m]], axis=0)
    pbit = (ig0 & m) == 0
    P = jnp.where(pbit, up, dn)
    mn = jnp.minimum(A, P)
    mx = jnp.maximum(A, P)
    dirn = (ig & k) == 0
    take_min = dirn == pbit
    return jnp.where(take_min, mn, mx)


def _tc_body(f_ref, obs_ref, w_ref, R_ref, d_ref, y_ref, *, S, B):
    t = pl.program_id(0)
    _R = S // _C
    NP = S.bit_length() - 1

    w_blk = w_ref[0]
    ones = jnp.ones((S,), jnp.float32)
    R_blk = jax.lax.dot_general(
        w_blk, ones, (((1,), (0,)), ((), ())),
        preferred_element_type=jnp.float32)
    rows = B // _C
    R_ref[pl.ds(t * rows, rows), :] = R_blk.reshape(rows, _C)

    ig_x = (jax.lax.broadcasted_iota(jnp.int32, (_R, _C), 0) * _C
            + jax.lax.broadcasted_iota(jnp.int32, (_R, _C), 1))
    ig_y = (jax.lax.broadcasted_iota(jnp.int32, (_C, _R), 0)
            + jax.lax.broadcasted_iota(jnp.int32, (_C, _R), 1) * _C)
    ig0_x = jax.lax.broadcasted_iota(jnp.int32, (_R, _C), 0)
    ig0_y = jax.lax.broadcasted_iota(jnp.int32, (_C, _R), 0)

    @pl.when(t == 0)
    def _():
        y_ref[...] = f_ref[...].T

    for p in range(NP):
        k = 1 << (p + 1)

        @pl.when(t == p)
        def _(k=k):
            jj = k // 2
            if jj >= _C:
                x = y_ref[...].T
                while jj >= _C:
                    x = _xchg_axis0(x, jj // _C, k, ig_x, ig0_x)
                    jj //= 2
                y_ref[...] = x.T
            y = y_ref[...]
            while jj >= 1:
                y = _xchg_axis0(y, jj, k, ig_y, ig0_y)
                jj //= 2
            y_ref[...] = y

    @pl.when(t == NP)
    def _():
        sf = y_ref[...].T
        a_le_b = (jax.lax.broadcasted_iota(jnp.int32, (_C, _C), 0)
                  <= jax.lax.broadcasted_iota(jnp.int32, (_C, _C), 1))
        L = jnp.where(a_le_b, 1.0, 0.0)
        cs_in = jax.lax.dot_general(
            sf, L, (((1,), (0,)), ((), ())),
            preferred_element_type=jnp.float32)
        rowsum = jax.lax.dot_general(
            sf, jnp.ones((_C,), jnp.float32), (((1,), (0,)), ((), ())),
            preferred_element_type=jnp.float32)
        a_lt_b = (jax.lax.broadcasted_iota(jnp.int32, (_R, _R), 0)
                  < jax.lax.broadcasted_iota(jnp.int32, (_R, _R), 1))
        U = jnp.where(a_lt_b, 1.0, 0.0)
        rp = jax.lax.dot_general(
            rowsum, U, (((0,), (0,)), ((), ())),
            preferred_element_type=jnp.float32)
        cs = cs_in + rp[:, None]
        ind = (sf > obs_ref[...]).astype(jnp.float32)
        d_ref[...] = (cs * (1.0 / S) - ind) ** 2


def _tc_call(f2d, obs2d, w_tc, *, S, B):
    nsteps = _RT // B
    return pl.pallas_call(
        functools.partial(_tc_body, S=S, B=B),
        grid=(nsteps,),
        in_specs=[
            pl.BlockSpec((S // _C, _C), lambda t: (0, 0)),
            pl.BlockSpec((S // _C, _C), lambda t: (0, 0)),
            pl.BlockSpec((1, B, S), lambda t: (0, t, 0)),
        ],
        out_specs=[
            pl.BlockSpec((_RT // _C, _C), lambda t: (0, 0)),
            pl.BlockSpec((S // _C, _C), lambda t: (0, 0)),
        ],
        out_shape=[
            jax.ShapeDtypeStruct((_RT // _C, _C), jnp.float32),
            jax.ShapeDtypeStruct((S // _C, _C), jnp.float32),
        ],
        scratch_shapes=[pltpu.VMEM((_C, S // _C), jnp.float32)],
    )(f2d, obs2d, w_tc)


def _sc_rowsum(w2d):
    mesh = plsc.VectorSubcoreMesh(core_axis_name="c", subcore_axis_name="s")
    info = plsc.get_sparse_core_info()
    nc = info.num_cores
    S = _S
    nchunk = S // 64

    @functools.partial(
        pl.kernel, mesh=mesh,
        out_type=jax.ShapeDtypeStruct(((_S - _RT) * 16,), jnp.float32),
        scratch_types=[
            pltpu.VMEM((2 * S,), jnp.float32),
            pltpu.VMEM((_RPW * 16,), jnp.float32),
            pltpu.SemaphoreType.DMA,
            pltpu.SemaphoreType.DMA,
        ],
    )
    def k(w_hbm, out_hbm, rowbuf, outbuf, sem0, sem1):
        wid = lax.axis_index("s") * nc + lax.axis_index("c")
        base = _RT + wid * _RPW
        sems = (sem0, sem1)

        def copy_row(r, buf_idx):
            for b in (0, 1):
                @pl.when(buf_idx == b)
                def _(b=b):
                    pltpu.make_async_copy(
                        w_hbm.at[base + r],
                        rowbuf.at[pl.ds(b * S, S)],
                        sems[b]).start()

        def wait_row(buf_idx):
            for b in (0, 1):
                @pl.when(buf_idx == b)
                def _(b=b):
                    pltpu.make_async_copy(
                        w_hbm.at[base],
                        rowbuf.at[pl.ds(b * S, S)],
                        sems[b]).wait()

        copy_row(jnp.int32(0), jnp.int32(0))

        def row_body(r, _):
            rb = lax.rem(r, 2)
            wait_row(rb)

            @pl.when(r + 1 < _RPW)
            def _():
                copy_row(r + 1, 1 - rb)

            def inner(i, accs):
                a0, a1, a2, a3 = accs
                off = rb * S + i * 64
                a0 = a0 + rowbuf[pl.ds(off, 16)]
                a1 = a1 + rowbuf[pl.ds(off + 16, 16)]
                a2 = a2 + rowbuf[pl.ds(off + 32, 16)]
                a3 = a3 + rowbuf[pl.ds(off + 48, 16)]
                return (a0, a1, a2, a3)

            z = jnp.zeros((16,), jnp.float32)
            a0, a1, a2, a3 = lax.fori_loop(0, nchunk, inner, (z, z, z, z))
            outbuf[pl.ds(r * 16, 16)] = (a0 + a1) + (a2 + a3)
            return 0

        lax.fori_loop(0, _RPW, row_body, 0)
        pltpu.sync_copy(outbuf, out_hbm.at[pl.ds(wid * _RPW * 16, _RPW * 16)])

    return k(w2d)


def _combine_body(Rtc_ref, Rsc_ref, d_ref, out_ref, *, S):
    rt_rows = _RT // _C
    d_tc = d_ref[:rt_rows, :]
    d_sc = d_ref[rt_rows:, :]
    Rsc = jnp.sum(Rsc_ref[...], axis=-1)
    tot = jnp.sum(Rtc_ref[...] * d_tc) + jnp.sum(Rsc * d_sc)
    out_ref[...] = jnp.reshape(tot / (S * S), (1, 1))


def _combine_call(R_tc, Rsc3d, d2d, *, S):
    return pl.pallas_call(
        functools.partial(_combine_body, S=S),
        out_shape=jax.ShapeDtypeStruct((1, 1), jnp.float32),
    )(R_tc, Rsc3d, d2d)


def kernel(forecast, observations, weights):
    S = _S
    f2d = forecast.reshape(S // _C, _C)
    obs2d = observations.reshape(S // _C, _C)
    w2d = weights.reshape(S, S)
    R_tc, d2d = _tc_call(f2d, obs2d, weights, S=S, B=256)
    Rsc_flat = _sc_rowsum(w2d)
    Rsc3d = Rsc_flat.reshape((S - _RT) // _C, _C, 16)
    out = _combine_call(R_tc, Rsc3d, d2d, S=S)
    return out[0, 0]


# speedup vs baseline: 1.1290x; 1.1290x over previous
import functools
import jax
import jax.numpy as jnp
from jax import lax
from jax.experimental import pallas as pl
from jax.experimental.pallas import tpu as pltpu
from jax.experimental.pallas import tpu_sc as plsc

_C = 128
_S = 8192
_RT = 6144
_NW = 32
_RPW = (_S - _RT) // _NW


def _xchg_axis0(A, m, k, ig, ig0):
    n0 = A.shape[0]
    up = jnp.concatenate([A[m:], A[:m]], axis=0)
    dn = jnp.concatenate([A[n0 - m:], A[:n0 - m]], axis=0)
    pbit = (ig0 & m) == 0
    P = jnp.where(pbit, up, dn)
    mn = jnp.minimum(A, P)
    mx = jnp.maximum(A, P)
    dirn = (ig & k) == 0
    take_min = dirn == pbit
    return jnp.where(take_min, mn, mx)


def _tc_body(f_ref, obs_ref, w_ref, R_ref, d_ref, y_ref, *, S, B):
    t = pl.program_id(0)
    _R = S // _C
    NP = S.bit_length() - 1

    w_blk = w_ref[0]
    ones = jnp.ones((S,), jnp.float32)
    R_blk = jax.lax.dot_general(
        w_blk, ones, (((1,), (0,)), ((), ())),
        preferred_element_type=jnp.float32)
    rows = B // _C
    R_ref[pl.ds(t * rows, rows), :] = R_blk.reshape(rows, _C)

    ig_x = (jax.lax.broadcasted_iota(jnp.int32, (_R, _C), 0) * _C
            + jax.lax.broadcasted_iota(jnp.int32, (_R, _C), 1))
    ig_y = (jax.lax.broadcasted_iota(jnp.int32, (_C, _R), 0)
            + jax.lax.broadcasted_iota(jnp.int32, (_C, _R), 1) * _C)
    ig0_x = jax.lax.broadcasted_iota(jnp.int32, (_R, _C), 0)
    ig0_y = jax.lax.broadcasted_iota(jnp.int32, (_C, _R), 0)

    @pl.when(t == 0)
    def _():
        y_ref[...] = f_ref[...].T

    for p in range(NP):
        k = 1 << (p + 1)

        @pl.when(t == p)
        def _(k=k):
            jj = k // 2
            if jj >= _C:
                x = y_ref[...].T
                while jj >= _C:
                    x = _xchg_axis0(x, jj // _C, k, ig_x, ig0_x)
                    jj //= 2
                y_ref[...] = x.T
            y = y_ref[...]
            while jj >= 1:
                y = _xchg_axis0(y, jj, k, ig_y, ig0_y)
                jj //= 2
            y_ref[...] = y

    @pl.when(t == NP)
    def _():
        sf = y_ref[...].T
        a_le_b = (jax.lax.broadcasted_iota(jnp.int32, (_C, _C), 0)
                  <= jax.lax.broadcasted_iota(jnp.int32, (_C, _C), 1))
        L = jnp.where(a_le_b, 1.0, 0.0)
        cs_in = jax.lax.dot_general(
            sf, L, (((1,), (0,)), ((), ())),
            preferred_element_type=jnp.float32)
        rowsum = jax.lax.dot_general(
            sf, jnp.ones((_C,), jnp.float32), (((1,), (0,)), ((), ())),
            preferred_element_type=jnp.float32)
        a_lt_b = (jax.lax.broadcasted_iota(jnp.int32, (_R, _R), 0)
                  < jax.lax.broadcasted_iota(jnp.int32, (_R, _R), 1))
        U = jnp.where(a_lt_b, 1.0, 0.0)
        rp = jax.lax.dot_general(
            rowsum, U, (((0,), (0,)), ((), ())),
            preferred_element_type=jnp.float32)
        cs = cs_in + rp[:, None]
        ind = (sf > obs_ref[...]).astype(jnp.float32)
        d_ref[...] = (cs * (1.0 / S) - ind) ** 2


def _tc_call(f2d, obs2d, w_tc, *, S, B):
    nsteps = _RT // B
    return pl.pallas_call(
        functools.partial(_tc_body, S=S, B=B),
        grid=(nsteps,),
        in_specs=[
            pl.BlockSpec((S // _C, _C), lambda t: (0, 0)),
            pl.BlockSpec((S // _C, _C), lambda t: (0, 0)),
            pl.BlockSpec((1, B, S), lambda t: (0, t, 0)),
        ],
        out_specs=[
            pl.BlockSpec((_RT // _C, _C), lambda t: (0, 0)),
            pl.BlockSpec((S // _C, _C), lambda t: (0, 0)),
        ],
        out_shape=[
            jax.ShapeDtypeStruct((_RT // _C, _C), jnp.float32),
            jax.ShapeDtypeStruct((S // _C, _C), jnp.float32),
        ],
        scratch_shapes=[pltpu.VMEM((_C, S // _C), jnp.float32)],
    )(f2d, obs2d, w_tc)


def _sc_rowsum(w2d):
    mesh = plsc.VectorSubcoreMesh(core_axis_name="c", subcore_axis_name="s")
    info = plsc.get_sparse_core_info()
    nc = info.num_cores
    S = _S
    nchunk = S // 64

    @functools.partial(
        pl.kernel, mesh=mesh,
        out_type=jax.ShapeDtypeStruct(((_S - _RT) * 16,), jnp.float32),
        scratch_types=[
            pltpu.VMEM((8, S), jnp.float32),
            pltpu.VMEM((_RPW * 16,), jnp.float32),
            pltpu.SemaphoreType.DMA,
            pltpu.SemaphoreType.DMA,
        ],
    )
    def k(w_hbm, out_hbm, rowbuf, outbuf, sem0, sem1):
        wid = lax.axis_index("s") * nc + lax.axis_index("c")
        gbase = _RT + wid * _RPW
        ngroups = _RPW // 4
        sems = (sem0, sem1)

        def copy_group(g, buf_idx):
            for b in (0, 1):
                @pl.when(buf_idx == b)
                def _(b=b):
                    pltpu.make_async_copy(
                        w_hbm.at[pl.ds(gbase + g * 4, 4), :],
                        rowbuf.at[pl.ds(b * 4, 4), :],
                        sems[b]).start()

        def wait_group(buf_idx):
            for b in (0, 1):
                @pl.when(buf_idx == b)
                def _(b=b):
                    pltpu.make_async_copy(
                        w_hbm.at[pl.ds(gbase, 4), :],
                        rowbuf.at[pl.ds(b * 4, 4), :],
                        sems[b]).wait()

        copy_group(jnp.int32(0), jnp.int32(0))

        def group_body(g, _):
            gb = lax.rem(g, 2)
            wait_group(gb)

            @pl.when(g + 1 < ngroups)
            def _():
                copy_group(g + 1, 1 - gb)

            for row in range(4):
                brow = gb * 4 + row

                def inner(i, accs):
                    off = i * 256
                    new = []
                    for u in range(16):
                        a = accs[u % 8]
                        a = a + rowbuf[brow, pl.ds(off + u * 16, 16)]
                        if u < 8:
                            new.append(a)
                        else:
                            new[u % 8] = a
                    return tuple(new)

                z = jnp.zeros((16,), jnp.float32)
                accs = lax.fori_loop(0, S // 256, inner, (z,) * 8)
                tot = accs[0]
                for u in range(1, 8):
                    tot = tot + accs[u]
                outbuf[pl.ds((g * 4 + row) * 16, 16)] = tot
            return 0

        lax.fori_loop(0, ngroups, group_body, 0)
        pltpu.sync_copy(outbuf, out_hbm.at[pl.ds(wid * _RPW * 16, _RPW * 16)])

    return k(w2d)


def _combine_body(Rtc_ref, Rsc_ref, d_ref, out_ref, *, S):
    rt_rows = _RT // _C
    d_tc = d_ref[:rt_rows, :]
    d_sc = d_ref[rt_rows:, :]
    Rsc = jnp.sum(Rsc_ref[...], axis=-1)
    tot = jnp.sum(Rtc_ref[...] * d_tc) + jnp.sum(Rsc * d_sc)
    out_ref[...] = jnp.reshape(tot / (S * S), (1, 1))


def _combine_call(R_tc, Rsc3d, d2d, *, S):
    return pl.pallas_call(
        functools.partial(_combine_body, S=S),
        out_shape=jax.ShapeDtypeStruct((1, 1), jnp.float32),
    )(R_tc, Rsc3d, d2d)


def kernel(forecast, observations, weights):
    S = _S
    f2d = forecast.reshape(S // _C, _C)
    obs2d = observations.reshape(S // _C, _C)
    w2d = weights.reshape(S, S)
    R_tc, d2d = _tc_call(f2d, obs2d, weights, S=S, B=256)
    Rsc_flat = _sc_rowsum(w2d)
    Rsc3d = Rsc_flat.reshape((S - _RT) // _C, _C, 16)
    out = _combine_call(R_tc, Rsc3d, d2d, S=S)
    return out[0, 0]


# speedup vs baseline: 1.1293x; 1.0003x over previous
import functools
import jax
import jax.numpy as jnp
from jax import lax
from jax.experimental import pallas as pl
from jax.experimental.pallas import tpu as pltpu
from jax.experimental.pallas import tpu_sc as plsc

_C = 128
_S = 8192
_RT = 6144
_NW = 32
_RPW = (_S - _RT) // _NW


def _xchg_axis0(A, m, k, ig, ig0):
    n0 = A.shape[0]
    up = jnp.concatenate([A[m:], A[:m]], axis=0)
    dn = jnp.concatenate([A[n0 - m:], A[:n0 - m]], axis=0)
    pbit = (ig0 & m) == 0
    P = jnp.where(pbit, up, dn)
    mn = jnp.minimum(A, P)
    mx = jnp.maximum(A, P)
    dirn = (ig & k) == 0
    take_min = dirn == pbit
    return jnp.where(take_min, mn, mx)


def _tc_body(f_ref, obs_ref, w_ref, R_ref, d_ref, y_ref, *, S, B):
    t = pl.program_id(0)
    _R = S // _C
    NP = S.bit_length() - 1

    w_blk = w_ref[0]
    ones = jnp.ones((S,), jnp.float32)
    R_blk = jax.lax.dot_general(
        w_blk, ones, (((1,), (0,)), ((), ())),
        preferred_element_type=jnp.float32)
    rows = B // _C
    R_ref[pl.ds(t * rows, rows), :] = R_blk.reshape(rows, _C)

    ig_x = (jax.lax.broadcasted_iota(jnp.int32, (_R, _C), 0) * _C
            + jax.lax.broadcasted_iota(jnp.int32, (_R, _C), 1))
    ig_y = (jax.lax.broadcasted_iota(jnp.int32, (_C, _R), 0)
            + jax.lax.broadcasted_iota(jnp.int32, (_C, _R), 1) * _C)
    ig0_x = jax.lax.broadcasted_iota(jnp.int32, (_R, _C), 0)
    ig0_y = jax.lax.broadcasted_iota(jnp.int32, (_C, _R), 0)

    @pl.when(t == 0)
    def _():
        y_ref[...] = f_ref[...].T

    for p in range(NP):
        k = 1 << (p + 1)

        @pl.when(t == p)
        def _(k=k):
            jj = k // 2
            if jj >= _C:
                x = y_ref[...].T
                while jj >= _C:
                    x = _xchg_axis0(x, jj // _C, k, ig_x, ig0_x)
                    jj //= 2
                y_ref[...] = x.T
            y = y_ref[...]
            while jj >= 1:
                y = _xchg_axis0(y, jj, k, ig_y, ig0_y)
                jj //= 2
            y_ref[...] = y

    @pl.when(t == NP)
    def _():
        sf = y_ref[...].T
        a_le_b = (jax.lax.broadcasted_iota(jnp.int32, (_C, _C), 0)
                  <= jax.lax.broadcasted_iota(jnp.int32, (_C, _C), 1))
        L = jnp.where(a_le_b, 1.0, 0.0)
        cs_in = jax.lax.dot_general(
            sf, L, (((1,), (0,)), ((), ())),
            preferred_element_type=jnp.float32)
        rowsum = jax.lax.dot_general(
            sf, jnp.ones((_C,), jnp.float32), (((1,), (0,)), ((), ())),
            preferred_element_type=jnp.float32)
        a_lt_b = (jax.lax.broadcasted_iota(jnp.int32, (_R, _R), 0)
                  < jax.lax.broadcasted_iota(jnp.int32, (_R, _R), 1))
        U = jnp.where(a_lt_b, 1.0, 0.0)
        rp = jax.lax.dot_general(
            rowsum, U, (((0,), (0,)), ((), ())),
            preferred_element_type=jnp.float32)
        cs = cs_in + rp[:, None]
        ind = (sf > obs_ref[...]).astype(jnp.float32)
        d_ref[...] = (cs * (1.0 / S) - ind) ** 2


def _tc_call(f2d, obs2d, w_tc, *, S, B):
    nsteps = _RT // B
    return pl.pallas_call(
        functools.partial(_tc_body, S=S, B=B),
        grid=(nsteps,),
        in_specs=[
            pl.BlockSpec((S // _C, _C), lambda t: (0, 0)),
            pl.BlockSpec((S // _C, _C), lambda t: (0, 0)),
            pl.BlockSpec((1, B, S), lambda t: (0, t, 0)),
        ],
        out_specs=[
            pl.BlockSpec((_RT // _C, _C), lambda t: (0, 0)),
            pl.BlockSpec((S // _C, _C), lambda t: (0, 0)),
        ],
        out_shape=[
            jax.ShapeDtypeStruct((_RT // _C, _C), jnp.float32),
            jax.ShapeDtypeStruct((S // _C, _C), jnp.float32),
        ],
        scratch_shapes=[pltpu.VMEM((_C, S // _C), jnp.float32)],
    )(f2d, obs2d, w_tc)


def _sc_rowsum(w2d):
    mesh = plsc.VectorSubcoreMesh(core_axis_name="c", subcore_axis_name="s")
    info = plsc.get_sparse_core_info()
    nc = info.num_cores
    S = _S
    nchunk = S // 64

    @functools.partial(
        pl.kernel, mesh=mesh,
        out_type=jax.ShapeDtypeStruct(((_S - _RT) * 16,), jnp.float32),
        scratch_types=[
            pltpu.VMEM((8, S), jnp.float32),
            pltpu.VMEM((_RPW * 16,), jnp.float32),
            pltpu.SemaphoreType.DMA,
            pltpu.SemaphoreType.DMA,
        ],
    )
    def k(w_hbm, out_hbm, rowbuf, outbuf, sem0, sem1):
        wid = lax.axis_index("s") * nc + lax.axis_index("c")
        gbase = _RT + wid * _RPW
        ngroups = _RPW // 4
        sems = (sem0, sem1)

        def copy_group(g, buf_idx):
            for b in (0, 1):
                @pl.when(buf_idx == b)
                def _(b=b):
                    pltpu.make_async_copy(
                        w_hbm.at[pl.ds(gbase + g * 4, 4), :],
                        rowbuf.at[pl.ds(b * 4, 4), :],
                        sems[b]).start()

        def wait_group(buf_idx):
            for b in (0, 1):
                @pl.when(buf_idx == b)
                def _(b=b):
                    pltpu.make_async_copy(
                        w_hbm.at[pl.ds(gbase, 4), :],
                        rowbuf.at[pl.ds(b * 4, 4), :],
                        sems[b]).wait()

        copy_group(jnp.int32(0), jnp.int32(0))

        def group_body(g, _):
            gb = lax.rem(g, 2)
            wait_group(gb)

            @pl.when(g + 1 < ngroups)
            def _():
                copy_group(g + 1, 1 - gb)

            for row in range(4):
                brow = gb * 4 + row

                def inner(i, accs):
                    off = i * 256
                    new = []
                    for u in range(16):
                        a = new[u % 8] if u >= 8 else accs[u % 8]
                        a = a + rowbuf[brow, pl.ds(off + u * 16, 16)]
                        if u < 8:
                            new.append(a)
                        else:
                            new[u % 8] = a
                    return tuple(new)

                z = jnp.zeros((16,), jnp.float32)
                accs = lax.fori_loop(0, S // 256, inner, (z,) * 8)
                tot = accs[0]
                for u in range(1, 8):
                    tot = tot + accs[u]
                outbuf[pl.ds((g * 4 + row) * 16, 16)] = tot
            return 0

        lax.fori_loop(0, ngroups, group_body, 0)
        pltpu.sync_copy(outbuf, out_hbm.at[pl.ds(wid * _RPW * 16, _RPW * 16)])

    return k(w2d)


def _combine_body(Rtc_ref, Rsc_ref, d_ref, out_ref, *, S):
    rt_rows = _RT // _C
    d_tc = d_ref[:rt_rows, :]
    d_sc = d_ref[rt_rows:, :]
    Rsc = jnp.sum(Rsc_ref[...], axis=-1)
    tot = jnp.sum(Rtc_ref[...] * d_tc) + jnp.sum(Rsc * d_sc)
    out_ref[...] = jnp.reshape(tot / (S * S), (1, 1))


def _combine_call(R_tc, Rsc3d, d2d, *, S):
    return pl.pallas_call(
        functools.partial(_combine_body, S=S),
        out_shape=jax.ShapeDtypeStruct((1, 1), jnp.float32),
    )(R_tc, Rsc3d, d2d)


def kernel(forecast, observations, weights):
    S = _S
    f2d = forecast.reshape(S // _C, _C)
    obs2d = observations.reshape(S // _C, _C)
    w2d = weights.reshape(S, S)
    R_tc, d2d = _tc_call(f2d, obs2d, weights, S=S, B=256)
    Rsc_flat = _sc_rowsum(w2d)
    Rsc3d = Rsc_flat.reshape((S - _RT) // _C, _C, 16)
    out = _combine_call(R_tc, Rsc3d, d2d, S=S)
    return out[0, 0]


# speedup vs baseline: 1.1303x; 1.0009x over previous
import functools
import jax
import jax.numpy as jnp
from jax import lax
from jax.experimental import pallas as pl
from jax.experimental.pallas import tpu as pltpu
from jax.experimental.pallas import tpu_sc as plsc

_C = 128
_S = 8192
_RT = 6144
_NW = 32
_RPW = (_S - _RT) // _NW


def _xchg_axis0(A, m, k, ig, ig0):
    n0 = A.shape[0]
    up = jnp.concatenate([A[m:], A[:m]], axis=0)
    dn = jnp.concatenate([A[n0 - m:], A[:n0 - m]], axis=0)
    pbit = (ig0 & m) == 0
    P = jnp.where(pbit, up, dn)
    mn = jnp.minimum(A, P)
    mx = jnp.maximum(A, P)
    dirn = (ig & k) == 0
    take_min = dirn == pbit
    return jnp.where(take_min, mn, mx)


def _tc_body(f_ref, obs_ref, w_ref, R_ref, d_ref, y_ref, *, S, B):
    t = pl.program_id(0)
    _R = S // _C
    NP = S.bit_length() - 1

    w_blk = w_ref[0]
    ones = jnp.ones((S,), jnp.float32)
    R_blk = jax.lax.dot_general(
        w_blk, ones, (((1,), (0,)), ((), ())),
        preferred_element_type=jnp.float32)
    rows = B // _C
    R_ref[pl.ds(t * rows, rows), :] = R_blk.reshape(rows, _C)

    ig_x = (jax.lax.broadcasted_iota(jnp.int32, (_R, _C), 0) * _C
            + jax.lax.broadcasted_iota(jnp.int32, (_R, _C), 1))
    ig_y = (jax.lax.broadcasted_iota(jnp.int32, (_C, _R), 0)
            + jax.lax.broadcasted_iota(jnp.int32, (_C, _R), 1) * _C)
    ig0_x = jax.lax.broadcasted_iota(jnp.int32, (_R, _C), 0)
    ig0_y = jax.lax.broadcasted_iota(jnp.int32, (_C, _R), 0)

    @pl.when(t == 0)
    def _():
        y_ref[...] = f_ref[...].T

    for p in range(NP):
        k = 1 << (p + 1)

        @pl.when(t == p)
        def _(k=k):
            jj = k // 2
            if jj >= _C:
                x = y_ref[...].T
                while jj >= _C:
                    x = _xchg_axis0(x, jj // _C, k, ig_x, ig0_x)
                    jj //= 2
                y_ref[...] = x.T
            y = y_ref[...]
            while jj >= 1:
                y = _xchg_axis0(y, jj, k, ig_y, ig0_y)
                jj //= 2
            y_ref[...] = y

    @pl.when(t == NP)
    def _():
        sf = y_ref[...].T
        a_le_b = (jax.lax.broadcasted_iota(jnp.int32, (_C, _C), 0)
                  <= jax.lax.broadcasted_iota(jnp.int32, (_C, _C), 1))
        L = jnp.where(a_le_b, 1.0, 0.0)
        cs_in = jax.lax.dot_general(
            sf, L, (((1,), (0,)), ((), ())),
            preferred_element_type=jnp.float32)
        rowsum = jax.lax.dot_general(
            sf, jnp.ones((_C,), jnp.float32), (((1,), (0,)), ((), ())),
            preferred_element_type=jnp.float32)
        a_lt_b = (jax.lax.broadcasted_iota(jnp.int32, (_R, _R), 0)
                  < jax.lax.broadcasted_iota(jnp.int32, (_R, _R), 1))
        U = jnp.where(a_lt_b, 1.0, 0.0)
        rp = jax.lax.dot_general(
            rowsum, U, (((0,), (0,)), ((), ())),
            preferred_element_type=jnp.float32)
        cs = cs_in + rp[:, None]
        ind = (sf > obs_ref[...]).astype(jnp.float32)
        d_ref[...] = (cs * (1.0 / S) - ind) ** 2


def _tc_call(f2d, obs2d, w_tc, *, S, B):
    nsteps = _RT // B
    return pl.pallas_call(
        functools.partial(_tc_body, S=S, B=B),
        grid=(nsteps,),
        in_specs=[
            pl.BlockSpec((S // _C, _C), lambda t: (0, 0)),
            pl.BlockSpec((S // _C, _C), lambda t: (0, 0)),
            pl.BlockSpec((1, B, S), lambda t: (0, t, 0)),
        ],
        out_specs=[
            pl.BlockSpec((_RT // _C, _C), lambda t: (0, 0)),
            pl.BlockSpec((S // _C, _C), lambda t: (0, 0)),
        ],
        out_shape=[
            jax.ShapeDtypeStruct((_RT // _C, _C), jnp.float32),
            jax.ShapeDtypeStruct((S // _C, _C), jnp.float32),
        ],
        scratch_shapes=[pltpu.VMEM((_C, S // _C), jnp.float32)],
    )(f2d, obs2d, w_tc)


def _sc_rowsum(w2d):
    mesh = plsc.VectorSubcoreMesh(core_axis_name="c", subcore_axis_name="s")
    info = plsc.get_sparse_core_info()
    nc = info.num_cores
    S = _S
    nchunk = S // 64

    @functools.partial(
        pl.kernel, mesh=mesh,
        out_type=jax.ShapeDtypeStruct(((_S - _RT) * 16,), jnp.float32),
        scratch_types=[
            pltpu.VMEM((8, S), jnp.float32),
            pltpu.VMEM((_RPW * 16,), jnp.float32),
            pltpu.SemaphoreType.DMA,
            pltpu.SemaphoreType.DMA,
        ],
    )
    def k(w_hbm, out_hbm, rowbuf, outbuf, sem0, sem1):
        wid = lax.axis_index("s") * nc + lax.axis_index("c")
        gbase = _RT + wid * _RPW
        ngroups = _RPW // 4
        sems = (sem0, sem1)

        def copy_group(g, buf_idx):
            for b in (0, 1):
                @pl.when(buf_idx == b)
                def _(b=b):
                    pltpu.make_async_copy(
                        w_hbm.at[pl.ds(gbase + g * 4, 4), :],
                        rowbuf.at[pl.ds(b * 4, 4), :],
                        sems[b]).start()

        def wait_group(buf_idx):
            for b in (0, 1):
                @pl.when(buf_idx == b)
                def _(b=b):
                    pltpu.make_async_copy(
                        w_hbm.at[pl.ds(gbase, 4), :],
                        rowbuf.at[pl.ds(b * 4, 4), :],
                        sems[b]).wait()

        copy_group(jnp.int32(0), jnp.int32(0))

        def group_body(g, _):
            gb = lax.rem(g, 2)
            wait_group(gb)

            @pl.when(g + 1 < ngroups)
            def _():
                copy_group(g + 1, 1 - gb)

            for row in range(4):
                brow = gb * 4 + row

                def inner(i, accs):
                    off = i * 256
                    new = []
                    for u in range(16):
                        a = new[u % 8] if u >= 8 else accs[u % 8]
                        a = a + rowbuf[brow, pl.ds(off + u * 16, 16)]
                        if u < 8:
                            new.append(a)
                        else:
                            new[u % 8] = a
                    return tuple(new)

                z = jnp.zeros((16,), jnp.float32)
                accs = lax.fori_loop(0, S // 256, inner, (z,) * 8)
                tot = accs[0]
                for u in range(1, 8):
                    tot = tot + accs[u]
                outbuf[pl.ds((g * 4 + row) * 16, 16)] = tot
            return 0

        lax.fori_loop(0, ngroups, group_body, 0)
        pltpu.sync_copy(outbuf, out_hbm.at[pl.ds(wid * _RPW * 16, _RPW * 16)])

    return k(w2d)


def _combine_body(Rtc_ref, Rsc_ref, d_ref, out_ref, *, S):
    rt_rows = _RT // _C
    d_tc = d_ref[:rt_rows, :]
    d_sc = d_ref[rt_rows:, :]
    Rsc = jnp.sum(Rsc_ref[...], axis=-1)
    tot = jnp.sum(Rtc_ref[...] * d_tc) + jnp.sum(Rsc * d_sc)
    out_ref[...] = jnp.reshape(tot / (S * S), (1, 1))


def _combine_call(R_tc, Rsc3d, d2d, *, S):
    return pl.pallas_call(
        functools.partial(_combine_body, S=S),
        out_shape=jax.ShapeDtypeStruct((1, 1), jnp.float32),
    )(R_tc, Rsc3d, d2d)


def kernel(forecast, observations, weights):
    S = _S
    f2d = forecast.reshape(S // _C, _C)
    obs2d = observations.reshape(S // _C, _C)
    w2d = weights.reshape(S, S)
    Rsc_flat = _sc_rowsum(w2d)
    R_tc, d2d = _tc_call(f2d, obs2d, weights, S=S, B=256)
    Rsc3d = Rsc_flat.reshape((S - _RT) // _C, _C, 16)
    out = _combine_call(R_tc, Rsc3d, d2d, S=S)
    return out[0, 0]


# speedup vs baseline: 1.4502x; 1.2830x over previous
import jax
import jax.numpy as jnp
from jax.experimental import pallas as pl
from jax.experimental.pallas import tpu as pltpu
from functools import partial

_C = 128


def _xchg_axis0(A, m, k, ig, ig0):
    n0 = A.shape[0]
    up = jnp.concatenate([A[m:], A[:m]], axis=0)
    dn = jnp.concatenate([A[n0 - m:], A[:n0 - m]], axis=0)
    pbit = (ig0 & m) == 0
    P = jnp.where(pbit, up, dn)
    mn = jnp.minimum(A, P)
    mx = jnp.maximum(A, P)
    dirn = (ig & k) == 0
    take_min = dirn == pbit
    return jnp.where(take_min, mn, mx)


def _crps_body(f_ref, obs_ref, w_ref, out_ref, y_ref, d_ref, R_ref, *, S, B):
    t = pl.program_id(0)
    nsteps = pl.num_programs(0)
    _R = S // _C
    NP = S.bit_length() - 1

    w_blk = w_ref[...]
    ones = jnp.ones((S,), jnp.float32)
    R_blk = jax.lax.dot_general(
        w_blk, ones, (((1,), (0,)), ((), ())),
        preferred_element_type=jnp.float32)
    rows = B // _C
    R_ref[pl.ds(t * rows, rows), :] = R_blk.reshape(rows, _C)

    ig_x = (jax.lax.broadcasted_iota(jnp.int32, (_R, _C), 0) * _C
            + jax.lax.broadcasted_iota(jnp.int32, (_R, _C), 1))
    ig_y = (jax.lax.broadcasted_iota(jnp.int32, (_C, _R), 0)
            + jax.lax.broadcasted_iota(jnp.int32, (_C, _R), 1) * _C)
    ig0_x = jax.lax.broadcasted_iota(jnp.int32, (_R, _C), 0)
    ig0_y = jax.lax.broadcasted_iota(jnp.int32, (_C, _R), 0)
    @pl.when(t == 0)
    def _():
        y_ref[...] = f_ref[...].T

    for p in range(NP):
        k = 1 << (p + 1)

        @pl.when(t == p)
        def _(k=k):
            jj = k // 2
            if jj >= _C:
                x = y_ref[...].T
                while jj >= _C:
                    x = _xchg_axis0(x, jj // _C, k, ig_x, ig0_x)
                    jj //= 2
                y_ref[...] = x.T
            y = y_ref[...]
            while jj >= 1:
                y = _xchg_axis0(y, jj, k, ig_y, ig0_y)
                jj //= 2
            y_ref[...] = y

    @pl.when(t == NP)
    def _():
        sf = y_ref[...].T
        a_le_b = (jax.lax.broadcasted_iota(jnp.int32, (_C, _C), 0)
                  <= jax.lax.broadcasted_iota(jnp.int32, (_C, _C), 1))
        L = jnp.where(a_le_b, 1.0, 0.0)
        cs_in = jax.lax.dot_general(
            sf, L, (((1,), (0,)), ((), ())),
            preferred_element_type=jnp.float32)
        rowsum = jax.lax.dot_general(
            sf, jnp.ones((_C,), jnp.float32), (((1,), (0,)), ((), ())),
            preferred_element_type=jnp.float32)
        a_lt_b = (jax.lax.broadcasted_iota(jnp.int32, (_R, _R), 0)
                  < jax.lax.broadcasted_iota(jnp.int32, (_R, _R), 1))
        U = jnp.where(a_lt_b, 1.0, 0.0)
        rp = jax.lax.dot_general(
            rowsum, U, (((0,), (0,)), ((), ())),
            preferred_element_type=jnp.float32)
        cs = cs_in + rp[:, None]
        ind = (sf > obs_ref[...]).astype(jnp.float32)
        d_ref[...] = (cs * (1.0 / S) - ind) ** 2

    @pl.when(t == nsteps - 1)
    def _():
        crps = jnp.sum(R_ref[...] * d_ref[...]) / (S * S)
        out_ref[...] = jnp.reshape(crps, (1, 1))


def _crps_pallas(forecast, observations, weights, *, B, interpret=False):
    S = forecast.size
    _R = S // _C
    nsteps = S // B
    assert nsteps > S.bit_length() - 1
    f2d = forecast.reshape(_R, _C)
    obs2d = observations.reshape(_R, _C)
    out = pl.pallas_call(
        partial(_crps_body, S=S, B=B),
        grid=(nsteps,),
        in_specs=[
            pl.BlockSpec((_R, _C), lambda t: (0, 0)),
            pl.BlockSpec((_R, _C), lambda t: (0, 0)),
            pl.BlockSpec((B, S), lambda t: (t, 0)),
        ],
        out_specs=pl.BlockSpec((1, 1), lambda t: (0, 0)),
        out_shape=jax.ShapeDtypeStruct((1, 1), jnp.float32),
        scratch_shapes=[
            pltpu.VMEM((_C, _R), jnp.float32),
            pltpu.VMEM((_R, _C), jnp.float32),
            pltpu.VMEM((_R, _C), jnp.float32),
        ],
        interpret=interpret,
    )(f2d, obs2d, weights.reshape(S, S))
    return out[0, 0]


def kernel(forecast, observations, weights):
    return _crps_pallas(forecast, observations, weights, B=256)
